# baseline (speedup 1.0000x reference)
import functools

import jax
import jax.numpy as jnp
from jax import lax
from jax.experimental import pallas as pl
from jax.experimental.pallas import tpu as pltpu
from jax.experimental.pallas import tpu_sc as plsc

N = 10000
E = 320000
D = 128
NC = 2
NS = 16
NW = NC * NS
E_PER_W = E // NW
CHUNK = 80
NCHUNK = E_PER_W // CHUNK
IB = 25
SUPER = NCHUNK // IB
N_PAD = 10240
ROWS_PER_S = N_PAD // NS
OUT_STEPS = ROWS_PER_S // CHUNK

_MESH = plsc.VectorSubcoreMesh(core_axis_name="c", subcore_axis_name="s")

_SC_SCRATCH = [
    pltpu.VMEM((IB, CHUNK), jnp.int32),
    pltpu.VMEM((IB, CHUNK), jnp.int32),
    pltpu.VMEM((CHUNK, D), jnp.float32),
    pltpu.VMEM((CHUNK, D), jnp.float32),
    pltpu.VMEM_SHARED((N_PAD, D), jnp.float32),
    pltpu.SemaphoreType.DMA,
]

_SC_SCRATCH_DB = [
    pltpu.VMEM((IB, CHUNK), jnp.int32),
    pltpu.VMEM((IB, CHUNK), jnp.int32),
    pltpu.VMEM((CHUNK, D), jnp.float32),
    pltpu.VMEM((CHUNK, D), jnp.float32),
    pltpu.VMEM_SHARED((N_PAD, D), jnp.float32),
    pltpu.SemaphoreType.DMA,
    pltpu.SemaphoreType.DMA,
]


def _sc_prologue(zeros_hbm, stage_v, tab_sh):
    sid = lax.axis_index("s")
    row0 = sid * ROWS_PER_S
    pltpu.sync_copy(zeros_hbm, stage_v)

    @pl.loop(0, OUT_STEPS)
    def _(k):
        pltpu.sync_copy(stage_v, tab_sh.at[pl.ds(row0 + k * CHUNK, CHUNK)])

    return row0


def _sc_epilogue(out_hbm, stage_v, tab_sh, row0):
    cid = lax.axis_index("c")

    @pl.loop(0, OUT_STEPS)
    def _(k):
        r = row0 + k * CHUNK
        pltpu.sync_copy(tab_sh.at[pl.ds(r, CHUNK)], stage_v)
        pltpu.sync_copy(stage_v, out_hbm.at[cid].at[pl.ds(r, CHUNK)])


@functools.partial(
    pl.kernel,
    out_type=jax.ShapeDtypeStruct((NC, N_PAD, D), jnp.float32),
    mesh=_MESH,
    scratch_types=_SC_SCRATCH,
)
def _sc_degree(dst_hbm, ones_hbm, zeros_hbm, out_hbm,
               src_v, dst_v, rows_v, stage_v, tab_sh, sem):
    cid = lax.axis_index("c")
    sid = lax.axis_index("s")
    wid = cid * NS + sid
    row0 = _sc_prologue(zeros_hbm, stage_v, tab_sh)
    pltpu.sync_copy(ones_hbm, rows_v)
    plsc.subcore_barrier()

    @pl.loop(0, SUPER)
    def _(g):
        pltpu.sync_copy(dst_hbm.at[wid].at[g], dst_v)

        @pl.loop(0, IB)
        def _(j):
            pltpu.sync_copy(rows_v, tab_sh.at[dst_v.at[j]], add=True)

    plsc.subcore_barrier()
    _sc_epilogue(out_hbm, stage_v, tab_sh, row0)


@functools.partial(
    pl.kernel,
    out_type=jax.ShapeDtypeStruct((NC, N_PAD, D), jnp.float32),
    mesh=_MESH,
    scratch_types=_SC_SCRATCH_DB,
)
def _sc_aggregate(h_hbm, src_hbm, dst_hbm, zeros_hbm, out_hbm,
                  src_v, dst_v, rows_a, rows_b, tab_sh,
                  sem_a, sem_b):
    cid = lax.axis_index("c")
    sid = lax.axis_index("s")
    wid = cid * NS + sid
    row0 = _sc_prologue(zeros_hbm, rows_a, tab_sh)
    plsc.subcore_barrier()

    @pl.loop(0, SUPER)
    def _(g):
        pltpu.sync_copy(src_hbm.at[wid].at[g], src_v)
        pltpu.sync_copy(dst_hbm.at[wid].at[g], dst_v)
        pltpu.async_copy(h_hbm.at[src_v.at[0]], rows_a, sem_a)

        @pl.loop(0, (IB - 1) // 2)
        def _(jj):
            j = jj * 2
            pltpu.async_copy(h_hbm.at[src_v.at[j + 1]], rows_b, sem_b)
            pltpu.make_async_copy(h_hbm.at[src_v.at[j]], rows_a, sem_a).wait()
            pltpu.sync_copy(rows_a, tab_sh.at[dst_v.at[j]], add=True)
            pltpu.async_copy(h_hbm.at[src_v.at[j + 2]], rows_a, sem_a)
            pltpu.make_async_copy(h_hbm.at[src_v.at[j + 1]], rows_b, sem_b).wait()
            pltpu.sync_copy(rows_b, tab_sh.at[dst_v.at[j + 1]], add=True)

        pltpu.make_async_copy(h_hbm.at[src_v.at[IB - 1]], rows_a, sem_a).wait()
        pltpu.sync_copy(rows_a, tab_sh.at[dst_v.at[IB - 1]], add=True)

    plsc.subcore_barrier()
    _sc_epilogue(out_hbm, rows_a, tab_sh, row0)


BLK = 1000


def _matmul_body(x_ref, w_ref, out_ref):
    out_ref[...] = jnp.dot(x_ref[...], w_ref[...],
                           preferred_element_type=jnp.float32,
                           precision=lax.Precision.HIGHEST)


def _tc_matmul(x, w):
    grid = (N // BLK,)
    return pl.pallas_call(
        _matmul_body,
        grid=grid,
        in_specs=[
            pl.BlockSpec((BLK, D), lambda i: (i, 0)),
            pl.BlockSpec((D, D), lambda i: (0, 0)),
        ],
        out_specs=pl.BlockSpec((BLK, D), lambda i: (i, 0)),
        out_shape=jax.ShapeDtypeStruct((N, D), jnp.float32),
    )(x, w)


def _prep_body(d0_ref, d1_ref, f1_ref, f1n_ref, nrm_ref):
    deg = d0_ref[...] + d1_ref[...] + 1.0
    nrm = lax.rsqrt(deg)
    nrm_ref[...] = nrm
    f1n_ref[...] = f1_ref[...] * nrm


def _tc_prep(d0, d1, f1):
    grid = (N // BLK,)
    return pl.pallas_call(
        _prep_body,
        grid=grid,
        in_specs=[
            pl.BlockSpec((BLK, D), lambda i: (i, 0)),
            pl.BlockSpec((BLK, D), lambda i: (i, 0)),
            pl.BlockSpec((BLK, D), lambda i: (i, 0)),
        ],
        out_specs=[
            pl.BlockSpec((BLK, D), lambda i: (i, 0)),
            pl.BlockSpec((BLK, D), lambda i: (i, 0)),
        ],
        out_shape=[
            jax.ShapeDtypeStruct((N, D), jnp.float32),
            jax.ShapeDtypeStruct((N, D), jnp.float32),
        ],
    )(d0, d1, f1)


def _layer_body(q0_ref, q1_ref, hn_ref, nrm_ref, out_ref):
    agg = q0_ref[...] + q1_ref[...] + hn_ref[...]
    s = nrm_ref[...]
    out_ref[...] = jnp.maximum(agg, 0.0) * (s * s)


def _tc_layer(q0, q1, hn, nrm):
    grid = (N // BLK,)
    return pl.pallas_call(
        _layer_body,
        grid=grid,
        in_specs=[
            pl.BlockSpec((BLK, D), lambda i: (i, 0)),
            pl.BlockSpec((BLK, D), lambda i: (i, 0)),
            pl.BlockSpec((BLK, D), lambda i: (i, 0)),
            pl.BlockSpec((BLK, D), lambda i: (i, 0)),
        ],
        out_specs=pl.BlockSpec((BLK, D), lambda i: (i, 0)),
        out_shape=jax.ShapeDtypeStruct((N, D), jnp.float32),
    )(q0, q1, hn, nrm)


def _shift1(x):
    return jnp.concatenate([x[:, 1:], x[:, 0:1]], axis=1)


def _conv_body(q0_ref, q1_ref, hn_ref, w_ref, nrm_ref, p_ref,
               hc_ref, cs_ref):
    agg = q0_ref[...] + q1_ref[...] + hn_ref[...]
    r = jnp.dot(agg, w_ref[...], preferred_element_type=jnp.float32,
                precision=lax.Precision.HIGHEST)
    h3 = jnp.maximum(r, 0.0) * nrm_ref[...]
    hs1 = _shift1(h3)
    hs2 = _shift1(hs1)
    t0 = p_ref[0] * h3 + p_ref[1] * hs1 + p_ref[2] * hs2
    t1 = p_ref[3] * h3 + p_ref[4] * hs1 + p_ref[5] * hs2
    lane = lax.broadcasted_iota(jnp.int32, h3.shape, 1)
    m = (lane < (D - 2)).astype(jnp.float32)
    hc = jnp.concatenate([(t0 + p_ref[6]) * m, (t1 + p_ref[7]) * m], axis=1)
    hc_ref[...] = hc

    @pl.when(pl.program_id(0) == 0)
    def _():
        cs_ref[...] = jnp.zeros_like(cs_ref)

    cs = jnp.sum(hc, axis=0, keepdims=True)
    cs_ref[...] += jnp.broadcast_to(cs, (8, 2 * D))


def _tc_conv(q0, q1, hn, w, nrm, params):
    grid = (N // BLK,)
    return pl.pallas_call(
        _conv_body,
        grid=grid,
        in_specs=[
            pl.BlockSpec((BLK, D), lambda i: (i, 0)),
            pl.BlockSpec((BLK, D), lambda i: (i, 0)),
            pl.BlockSpec((BLK, D), lambda i: (i, 0)),
            pl.BlockSpec((D, D), lambda i: (0, 0)),
            pl.BlockSpec((BLK, D), lambda i: (i, 0)),
            pl.BlockSpec(memory_space=pltpu.SMEM),
        ],
        out_specs=[
            pl.BlockSpec((BLK, 2 * D), lambda i: (i, 0)),
            pl.BlockSpec((8, 2 * D), lambda i: (0, 0)),
        ],
        out_shape=[
            jax.ShapeDtypeStruct((N, 2 * D), jnp.float32),
            jax.ShapeDtypeStruct((8, 2 * D), jnp.float32),
        ],
    )(q0, q1, hn, w, nrm, params)


def _radius_body(hc_ref, cs_ref, rr_ref, out_ref):
    o = cs_ref[0:1, :] * (1.0 / N)
    d = hc_ref[...] - o + 1e-6
    lane = lax.broadcasted_iota(jnp.int32, d.shape, 1)
    m = ((lane % D) < (D - 2)).astype(jnp.float32)
    d = d * m
    r = jnp.sqrt(jnp.sum(d * d, axis=1, keepdims=True))
    out_ref[...] = jnp.clip(r - rr_ref[0], 0.0001, 1.0 - 0.0001)


def _tc_radius(hc, cs, ref_radius):
    grid = (N // BLK,)
    return pl.pallas_call(
        _radius_body,
        grid=grid,
        in_specs=[
            pl.BlockSpec((BLK, 2 * D), lambda i: (i, 0)),
            pl.BlockSpec((8, 2 * D), lambda i: (0, 0)),
            pl.BlockSpec(memory_space=pltpu.SMEM),
        ],
        out_specs=pl.BlockSpec((BLK, 1), lambda i: (i, 0)),
        out_shape=jax.ShapeDtypeStruct((N, 1), jnp.float32),
    )(hc, cs, ref_radius)


def kernel(feat, edge_index, W1, W2, conv_w, conv_b, ref_radius):
    src = edge_index[0].reshape(NW, SUPER, IB, CHUNK)
    dst = edge_index[1].reshape(NW, SUPER, IB, CHUNK)
    ones128 = jnp.ones((CHUNK, D), jnp.float32)
    zeros128 = jnp.zeros((CHUNK, D), jnp.float32)
    params = jnp.concatenate([conv_w.reshape(6), conv_b]).astype(jnp.float32)

    f1 = _tc_matmul(feat.astype(jnp.float32), W1)
    degp = _sc_degree(dst, ones128, zeros128)
    f1n, nrm = _tc_prep(degp[0], degp[1], f1)

    p = _sc_aggregate(f1n, src, dst, zeros128)
    hn2 = _tc_layer(p[0], p[1], f1n, nrm)

    q = _sc_aggregate(hn2, src, dst, zeros128)
    hc, cs = _tc_conv(q[0], q[1], hn2, W2, nrm, params)

    dis = _tc_radius(hc, cs, ref_radius)
    return (dis.reshape(N), ref_radius)


# speedup vs baseline: 1.0236x; 1.0236x over previous
import functools

import jax
import jax.numpy as jnp
from jax import lax
from jax.experimental import pallas as pl
from jax.experimental.pallas import tpu as pltpu
from jax.experimental.pallas import tpu_sc as plsc

N = 10000
E = 320000
D = 128
NC = 2
NS = 16
NW = NC * NS
E_PER_W = E // NW
CHUNK = 80
NCHUNK = E_PER_W // CHUNK
IB = 25
SUPER = NCHUNK // IB
N_PAD = 10240
ROWS_PER_S = N_PAD // NS
OUT_STEPS = ROWS_PER_S // CHUNK

_MESH = plsc.VectorSubcoreMesh(core_axis_name="c", subcore_axis_name="s")

_SC_SCRATCH = [
    pltpu.VMEM((IB, CHUNK), jnp.int32),
    pltpu.VMEM((IB, CHUNK), jnp.int32),
    pltpu.VMEM((CHUNK, D), jnp.float32),
    pltpu.VMEM((CHUNK, D), jnp.float32),
    pltpu.VMEM_SHARED((N_PAD, D), jnp.float32),
    pltpu.SemaphoreType.DMA,
]

_SC_SCRATCH_DB = [
    pltpu.VMEM((2 * IB, CHUNK), jnp.int32),
    pltpu.VMEM((2 * IB, CHUNK), jnp.int32),
    pltpu.VMEM((CHUNK, D), jnp.float32),
    pltpu.VMEM((CHUNK, D), jnp.float32),
    pltpu.VMEM_SHARED((N_PAD, D), jnp.float32),
    pltpu.SemaphoreType.DMA,
    pltpu.SemaphoreType.DMA,
    pltpu.SemaphoreType.DMA,
]


def _sc_prologue(zeros_hbm, stage_v, tab_sh):
    sid = lax.axis_index("s")
    row0 = sid * ROWS_PER_S
    pltpu.sync_copy(zeros_hbm, stage_v)

    @pl.loop(0, OUT_STEPS)
    def _(k):
        pltpu.sync_copy(stage_v, tab_sh.at[pl.ds(row0 + k * CHUNK, CHUNK)])

    return row0


def _sc_epilogue(out_hbm, stage_v, tab_sh, row0):
    cid = lax.axis_index("c")

    @pl.loop(0, OUT_STEPS)
    def _(k):
        r = row0 + k * CHUNK
        pltpu.sync_copy(tab_sh.at[pl.ds(r, CHUNK)], stage_v)
        pltpu.sync_copy(stage_v, out_hbm.at[cid].at[pl.ds(r, CHUNK)])


@functools.partial(
    pl.kernel,
    out_type=jax.ShapeDtypeStruct((NC, N_PAD, D), jnp.float32),
    mesh=_MESH,
    scratch_types=_SC_SCRATCH,
)
def _sc_degree(dst_hbm, ones_hbm, zeros_hbm, out_hbm,
               src_v, dst_v, rows_v, stage_v, tab_sh, sem):
    cid = lax.axis_index("c")
    sid = lax.axis_index("s")
    wid = cid * NS + sid
    row0 = _sc_prologue(zeros_hbm, stage_v, tab_sh)
    pltpu.sync_copy(ones_hbm, rows_v)
    plsc.subcore_barrier()

    @pl.loop(0, SUPER)
    def _(g):
        pltpu.sync_copy(dst_hbm.at[wid].at[g], dst_v)

        @pl.loop(0, IB)
        def _(j):
            pltpu.sync_copy(rows_v, tab_sh.at[dst_v.at[j]], add=True)

    plsc.subcore_barrier()
    _sc_epilogue(out_hbm, stage_v, tab_sh, row0)


@functools.partial(
    pl.kernel,
    out_type=jax.ShapeDtypeStruct((NC, N_PAD, D), jnp.float32),
    mesh=_MESH,
    scratch_types=_SC_SCRATCH_DB,
)
def _sc_aggregate(h_hbm, ei_hbm, zeros_hbm, out_hbm,
                  idx_a, idx_b, rows_a, rows_b, tab_sh,
                  sem_a, sem_b, sem_i):
    cid = lax.axis_index("c")
    sid = lax.axis_index("s")
    wid = cid * NS + sid
    row0 = _sc_prologue(zeros_hbm, rows_a, tab_sh)
    plsc.subcore_barrier()

    def super_chunk(g, ibuf, nbuf):
        src = ibuf
        pltpu.async_copy(h_hbm.at[src.at[0]], rows_a, sem_a)
        if g + 1 < SUPER:
            pltpu.async_copy(ei_hbm.at[wid].at[g + 1], nbuf, sem_i)

        @pl.loop(0, (IB - 1) // 2)
        def _(jj):
            j = jj * 2
            pltpu.async_copy(h_hbm.at[src.at[j + 1]], rows_b, sem_b)
            pltpu.make_async_copy(h_hbm.at[src.at[j]], rows_a, sem_a).wait()
            pltpu.sync_copy(rows_a, tab_sh.at[src.at[IB + j]], add=True)
            pltpu.async_copy(h_hbm.at[src.at[j + 2]], rows_a, sem_a)
            pltpu.make_async_copy(h_hbm.at[src.at[j + 1]], rows_b, sem_b).wait()
            pltpu.sync_copy(rows_b, tab_sh.at[src.at[IB + j + 1]], add=True)

        pltpu.make_async_copy(h_hbm.at[src.at[IB - 1]], rows_a, sem_a).wait()
        pltpu.sync_copy(rows_a, tab_sh.at[src.at[2 * IB - 1]], add=True)
        if g + 1 < SUPER:
            pltpu.make_async_copy(ei_hbm.at[wid].at[g + 1], nbuf, sem_i).wait()

    pltpu.sync_copy(ei_hbm.at[wid].at[0], idx_a)
    for g in range(SUPER):
        super_chunk(g, idx_a if g % 2 == 0 else idx_b,
                    idx_b if g % 2 == 0 else idx_a)

    plsc.subcore_barrier()
    _sc_epilogue(out_hbm, rows_a, tab_sh, row0)


BLK = 1000


def _matmul_body(x_ref, w_ref, out_ref):
    out_ref[...] = jnp.dot(x_ref[...], w_ref[...],
                           preferred_element_type=jnp.float32,
                           precision=lax.Precision.HIGHEST)


def _tc_matmul(x, w):
    grid = (N // BLK,)
    return pl.pallas_call(
        _matmul_body,
        grid=grid,
        in_specs=[
            pl.BlockSpec((BLK, D), lambda i: (i, 0)),
            pl.BlockSpec((D, D), lambda i: (0, 0)),
        ],
        out_specs=pl.BlockSpec((BLK, D), lambda i: (i, 0)),
        out_shape=jax.ShapeDtypeStruct((N, D), jnp.float32),
    )(x, w)


def _prep_body(d0_ref, d1_ref, f1_ref, f1n_ref, nrm_ref):
    deg = d0_ref[...] + d1_ref[...] + 1.0
    nrm = lax.rsqrt(deg)
    nrm_ref[...] = nrm
    f1n_ref[...] = f1_ref[...] * nrm


def _tc_prep(d0, d1, f1):
    grid = (N // BLK,)
    return pl.pallas_call(
        _prep_body,
        grid=grid,
        in_specs=[
            pl.BlockSpec((BLK, D), lambda i: (i, 0)),
            pl.BlockSpec((BLK, D), lambda i: (i, 0)),
            pl.BlockSpec((BLK, D), lambda i: (i, 0)),
        ],
        out_specs=[
            pl.BlockSpec((BLK, D), lambda i: (i, 0)),
            pl.BlockSpec((BLK, D), lambda i: (i, 0)),
        ],
        out_shape=[
            jax.ShapeDtypeStruct((N, D), jnp.float32),
            jax.ShapeDtypeStruct((N, D), jnp.float32),
        ],
    )(d0, d1, f1)


def _layer_body(q0_ref, q1_ref, hn_ref, nrm_ref, out_ref):
    agg = q0_ref[...] + q1_ref[...] + hn_ref[...]
    s = nrm_ref[...]
    out_ref[...] = jnp.maximum(agg, 0.0) * (s * s)


def _tc_layer(q0, q1, hn, nrm):
    grid = (N // BLK,)
    return pl.pallas_call(
        _layer_body,
        grid=grid,
        in_specs=[
            pl.BlockSpec((BLK, D), lambda i: (i, 0)),
            pl.BlockSpec((BLK, D), lambda i: (i, 0)),
            pl.BlockSpec((BLK, D), lambda i: (i, 0)),
            pl.BlockSpec((BLK, D), lambda i: (i, 0)),
        ],
        out_specs=pl.BlockSpec((BLK, D), lambda i: (i, 0)),
        out_shape=jax.ShapeDtypeStruct((N, D), jnp.float32),
    )(q0, q1, hn, nrm)


def _shift1(x):
    return jnp.concatenate([x[:, 1:], x[:, 0:1]], axis=1)


def _conv_body(q0_ref, q1_ref, hn_ref, w_ref, nrm_ref, p_ref,
               hc_ref, cs_ref):
    agg = q0_ref[...] + q1_ref[...] + hn_ref[...]
    r = jnp.dot(agg, w_ref[...], preferred_element_type=jnp.float32,
                precision=lax.Precision.HIGHEST)
    h3 = jnp.maximum(r, 0.0) * nrm_ref[...]
    hs1 = _shift1(h3)
    hs2 = _shift1(hs1)
    t0 = p_ref[0] * h3 + p_ref[1] * hs1 + p_ref[2] * hs2
    t1 = p_ref[3] * h3 + p_ref[4] * hs1 + p_ref[5] * hs2
    lane = lax.broadcasted_iota(jnp.int32, h3.shape, 1)
    m = (lane < (D - 2)).astype(jnp.float32)
    hc = jnp.concatenate([(t0 + p_ref[6]) * m, (t1 + p_ref[7]) * m], axis=1)
    hc_ref[...] = hc

    @pl.when(pl.program_id(0) == 0)
    def _():
        cs_ref[...] = jnp.zeros_like(cs_ref)

    cs = jnp.sum(hc, axis=0, keepdims=True)
    cs_ref[...] += jnp.broadcast_to(cs, (8, 2 * D))


def _tc_conv(q0, q1, hn, w, nrm, params):
    grid = (N // BLK,)
    return pl.pallas_call(
        _conv_body,
        grid=grid,
        in_specs=[
            pl.BlockSpec((BLK, D), lambda i: (i, 0)),
            pl.BlockSpec((BLK, D), lambda i: (i, 0)),
            pl.BlockSpec((BLK, D), lambda i: (i, 0)),
            pl.BlockSpec((D, D), lambda i: (0, 0)),
            pl.BlockSpec((BLK, D), lambda i: (i, 0)),
            pl.BlockSpec(memory_space=pltpu.SMEM),
        ],
        out_specs=[
            pl.BlockSpec((BLK, 2 * D), lambda i: (i, 0)),
            pl.BlockSpec((8, 2 * D), lambda i: (0, 0)),
        ],
        out_shape=[
            jax.ShapeDtypeStruct((N, 2 * D), jnp.float32),
            jax.ShapeDtypeStruct((8, 2 * D), jnp.float32),
        ],
    )(q0, q1, hn, w, nrm, params)


def _radius_body(hc_ref, cs_ref, rr_ref, out_ref):
    o = cs_ref[0:1, :] * (1.0 / N)
    d = hc_ref[...] - o + 1e-6
    lane = lax.broadcasted_iota(jnp.int32, d.shape, 1)
    m = ((lane % D) < (D - 2)).astype(jnp.float32)
    d = d * m
    r = jnp.sqrt(jnp.sum(d * d, axis=1, keepdims=True))
    out_ref[...] = jnp.clip(r - rr_ref[0], 0.0001, 1.0 - 0.0001)


def _tc_radius(hc, cs, ref_radius):
    grid = (N // BLK,)
    return pl.pallas_call(
        _radius_body,
        grid=grid,
        in_specs=[
            pl.BlockSpec((BLK, 2 * D), lambda i: (i, 0)),
            pl.BlockSpec((8, 2 * D), lambda i: (0, 0)),
            pl.BlockSpec(memory_space=pltpu.SMEM),
        ],
        out_specs=pl.BlockSpec((BLK, 1), lambda i: (i, 0)),
        out_shape=jax.ShapeDtypeStruct((N, 1), jnp.float32),
    )(hc, cs, ref_radius)


def kernel(feat, edge_index, W1, W2, conv_w, conv_b, ref_radius):
    src = edge_index[0].reshape(NW, SUPER, IB, CHUNK)
    dst = edge_index[1].reshape(NW, SUPER, IB, CHUNK)
    ei = jnp.stack([src, dst], axis=2).reshape(NW, SUPER, 2 * IB, CHUNK)
    ones128 = jnp.ones((CHUNK, D), jnp.float32)
    zeros128 = jnp.zeros((CHUNK, D), jnp.float32)
    params = jnp.concatenate([conv_w.reshape(6), conv_b]).astype(jnp.float32)

    f1 = _tc_matmul(feat.astype(jnp.float32), W1)
    degp = _sc_degree(dst, ones128, zeros128)
    f1n, nrm = _tc_prep(degp[0], degp[1], f1)

    p = _sc_aggregate(f1n, ei, zeros128)
    hn2 = _tc_layer(p[0], p[1], f1n, nrm)

    q = _sc_aggregate(hn2, ei, zeros128)
    hc, cs = _tc_conv(q[0], q[1], hn2, W2, nrm, params)

    dis = _tc_radius(hc, cs, ref_radius)
    return (dis.reshape(N), ref_radius)


# speedup vs baseline: 1.1205x; 1.0947x over previous
import functools

import jax
import jax.numpy as jnp
from jax import lax
from jax.experimental import pallas as pl
from jax.experimental.pallas import tpu as pltpu
from jax.experimental.pallas import tpu_sc as plsc

N = 10000
E = 320000
D = 128
NC = 2
NS = 16
NW = NC * NS
E_PER_W = E // NW
CHUNK = 80
NCHUNK = E_PER_W // CHUNK
IB = 25
SUPER = NCHUNK // IB
N_PAD = 10240
ROWS_PER_S = N_PAD // NS
OUT_STEPS = ROWS_PER_S // CHUNK

_MESH = plsc.VectorSubcoreMesh(core_axis_name="c", subcore_axis_name="s")

_SC_SCRATCH = [
    pltpu.VMEM((IB, CHUNK), jnp.int32),
    pltpu.VMEM((IB, CHUNK), jnp.int32),
    pltpu.VMEM((CHUNK, D), jnp.float32),
    pltpu.VMEM((CHUNK, D), jnp.float32),
    pltpu.VMEM_SHARED((N_PAD, D), jnp.float32),
    pltpu.SemaphoreType.DMA,
]

_SC_SCRATCH_DB = [
    pltpu.VMEM((2 * IB, CHUNK), jnp.int32),
    pltpu.VMEM((2 * IB, CHUNK), jnp.int32),
    pltpu.VMEM((CHUNK, D), jnp.float32),
    pltpu.VMEM((CHUNK, D), jnp.float32),
    pltpu.VMEM((CHUNK, D), jnp.float32),
    pltpu.VMEM_SHARED((N_PAD, D), jnp.float32),
    pltpu.SemaphoreType.DMA,
    pltpu.SemaphoreType.DMA,
    pltpu.SemaphoreType.DMA,
    pltpu.SemaphoreType.DMA,
]


def _sc_prologue(zeros_hbm, stage_v, tab_sh):
    sid = lax.axis_index("s")
    row0 = sid * ROWS_PER_S
    pltpu.sync_copy(zeros_hbm, stage_v)

    @pl.loop(0, OUT_STEPS)
    def _(k):
        pltpu.sync_copy(stage_v, tab_sh.at[pl.ds(row0 + k * CHUNK, CHUNK)])

    return row0


def _sc_epilogue(out_hbm, stage_v, tab_sh, row0):
    cid = lax.axis_index("c")

    @pl.loop(0, OUT_STEPS)
    def _(k):
        r = row0 + k * CHUNK
        pltpu.sync_copy(tab_sh.at[pl.ds(r, CHUNK)], stage_v)
        pltpu.sync_copy(stage_v, out_hbm.at[cid].at[pl.ds(r, CHUNK)])


@functools.partial(
    pl.kernel,
    out_type=jax.ShapeDtypeStruct((NC, N_PAD, D), jnp.float32),
    mesh=_MESH,
    scratch_types=_SC_SCRATCH,
)
def _sc_degree(dst_hbm, ones_hbm, zeros_hbm, out_hbm,
               src_v, dst_v, rows_v, stage_v, tab_sh, sem):
    cid = lax.axis_index("c")
    sid = lax.axis_index("s")
    wid = cid * NS + sid
    row0 = _sc_prologue(zeros_hbm, stage_v, tab_sh)
    pltpu.sync_copy(ones_hbm, rows_v)
    plsc.subcore_barrier()

    @pl.loop(0, SUPER)
    def _(g):
        pltpu.sync_copy(dst_hbm.at[wid].at[g], dst_v)

        @pl.loop(0, IB)
        def _(j):
            pltpu.sync_copy(rows_v, tab_sh.at[dst_v.at[j]], add=True)

    plsc.subcore_barrier()
    _sc_epilogue(out_hbm, stage_v, tab_sh, row0)


@functools.partial(
    pl.kernel,
    out_type=jax.ShapeDtypeStruct((NC, N_PAD, D), jnp.float32),
    mesh=_MESH,
    scratch_types=_SC_SCRATCH_DB,
)
def _sc_aggregate(h_hbm, ei_hbm, zeros_hbm, out_hbm,
                  idx_a, idx_b, rows_a, rows_b, rows_c, tab_sh,
                  sem_a, sem_b, sem_c, sem_i):
    cid = lax.axis_index("c")
    sid = lax.axis_index("s")
    wid = cid * NS + sid
    row0 = _sc_prologue(zeros_hbm, rows_a, tab_sh)
    plsc.subcore_barrier()

    def super_chunk(g, ibuf, nbuf):
        src = ibuf
        pltpu.async_copy(h_hbm.at[src.at[0]], rows_a, sem_a)
        pltpu.async_copy(h_hbm.at[src.at[1]], rows_b, sem_b)
        if g + 1 < SUPER:
            pltpu.async_copy(ei_hbm.at[wid].at[g + 1], nbuf, sem_i)

        @pl.loop(0, (IB - 1) // 3)
        def _(jj):
            j = jj * 3
            pltpu.async_copy(h_hbm.at[src.at[j + 2]], rows_c, sem_c)
            pltpu.make_async_copy(h_hbm.at[src.at[j]], rows_a, sem_a).wait()
            pltpu.sync_copy(rows_a, tab_sh.at[src.at[IB + j]], add=True)
            pltpu.async_copy(h_hbm.at[src.at[j + 3]], rows_a, sem_a)
            pltpu.make_async_copy(h_hbm.at[src.at[j + 1]], rows_b, sem_b).wait()
            pltpu.sync_copy(rows_b, tab_sh.at[src.at[IB + j + 1]], add=True)

            @pl.when(j + 4 < IB)
            def _():
                pltpu.async_copy(h_hbm.at[src.at[j + 4]], rows_b, sem_b)

            pltpu.make_async_copy(h_hbm.at[src.at[j + 2]], rows_c, sem_c).wait()
            pltpu.sync_copy(rows_c, tab_sh.at[src.at[IB + j + 2]], add=True)

        pltpu.make_async_copy(h_hbm.at[src.at[IB - 1]], rows_a, sem_a).wait()
        pltpu.sync_copy(rows_a, tab_sh.at[src.at[2 * IB - 1]], add=True)
        if g + 1 < SUPER:
            pltpu.make_async_copy(ei_hbm.at[wid].at[g + 1], nbuf, sem_i).wait()

    pltpu.sync_copy(ei_hbm.at[wid].at[0], idx_a)
    for g in range(SUPER):
        super_chunk(g, idx_a if g % 2 == 0 else idx_b,
                    idx_b if g % 2 == 0 else idx_a)

    plsc.subcore_barrier()
    _sc_epilogue(out_hbm, rows_a, tab_sh, row0)


BLK = 1000


def _matmul_body(x_ref, w_ref, out_ref):
    out_ref[...] = jnp.dot(x_ref[...], w_ref[...],
                           preferred_element_type=jnp.float32,
                           precision=lax.Precision.HIGHEST)


def _tc_matmul(x, w):
    grid = (N // BLK,)
    return pl.pallas_call(
        _matmul_body,
        grid=grid,
        in_specs=[
            pl.BlockSpec((BLK, D), lambda i: (i, 0)),
            pl.BlockSpec((D, D), lambda i: (0, 0)),
        ],
        out_specs=pl.BlockSpec((BLK, D), lambda i: (i, 0)),
        out_shape=jax.ShapeDtypeStruct((N, D), jnp.float32),
    )(x, w)


def _prep_body(d0_ref, d1_ref, f1_ref, f1n_ref, nrm_ref):
    deg = d0_ref[...] + d1_ref[...] + 1.0
    nrm = lax.rsqrt(deg)
    nrm_ref[...] = nrm
    f1n_ref[...] = f1_ref[...] * nrm


def _tc_prep(d0, d1, f1):
    grid = (N // BLK,)
    return pl.pallas_call(
        _prep_body,
        grid=grid,
        in_specs=[
            pl.BlockSpec((BLK, D), lambda i: (i, 0)),
            pl.BlockSpec((BLK, D), lambda i: (i, 0)),
            pl.BlockSpec((BLK, D), lambda i: (i, 0)),
        ],
        out_specs=[
            pl.BlockSpec((BLK, D), lambda i: (i, 0)),
            pl.BlockSpec((BLK, D), lambda i: (i, 0)),
        ],
        out_shape=[
            jax.ShapeDtypeStruct((N, D), jnp.float32),
            jax.ShapeDtypeStruct((N, D), jnp.float32),
        ],
    )(d0, d1, f1)


def _layer_body(q0_ref, q1_ref, hn_ref, nrm_ref, out_ref):
    agg = q0_ref[...] + q1_ref[...] + hn_ref[...]
    s = nrm_ref[...]
    out_ref[...] = jnp.maximum(agg, 0.0) * (s * s)


def _tc_layer(q0, q1, hn, nrm):
    grid = (N // BLK,)
    return pl.pallas_call(
        _layer_body,
        grid=grid,
        in_specs=[
            pl.BlockSpec((BLK, D), lambda i: (i, 0)),
            pl.BlockSpec((BLK, D), lambda i: (i, 0)),
            pl.BlockSpec((BLK, D), lambda i: (i, 0)),
            pl.BlockSpec((BLK, D), lambda i: (i, 0)),
        ],
        out_specs=pl.BlockSpec((BLK, D), lambda i: (i, 0)),
        out_shape=jax.ShapeDtypeStruct((N, D), jnp.float32),
    )(q0, q1, hn, nrm)


def _shift1(x):
    return jnp.concatenate([x[:, 1:], x[:, 0:1]], axis=1)


def _conv_body(q0_ref, q1_ref, hn_ref, w_ref, nrm_ref, p_ref,
               hc_ref, cs_ref):
    agg = q0_ref[...] + q1_ref[...] + hn_ref[...]
    r = jnp.dot(agg, w_ref[...], preferred_element_type=jnp.float32,
                precision=lax.Precision.HIGHEST)
    h3 = jnp.maximum(r, 0.0) * nrm_ref[...]
    hs1 = _shift1(h3)
    hs2 = _shift1(hs1)
    t0 = p_ref[0] * h3 + p_ref[1] * hs1 + p_ref[2] * hs2
    t1 = p_ref[3] * h3 + p_ref[4] * hs1 + p_ref[5] * hs2
    lane = lax.broadcasted_iota(jnp.int32, h3.shape, 1)
    m = (lane < (D - 2)).astype(jnp.float32)
    hc = jnp.concatenate([(t0 + p_ref[6]) * m, (t1 + p_ref[7]) * m], axis=1)
    hc_ref[...] = hc

    @pl.when(pl.program_id(0) == 0)
    def _():
        cs_ref[...] = jnp.zeros_like(cs_ref)

    cs = jnp.sum(hc, axis=0, keepdims=True)
    cs_ref[...] += jnp.broadcast_to(cs, (8, 2 * D))


def _tc_conv(q0, q1, hn, w, nrm, params):
    grid = (N // BLK,)
    return pl.pallas_call(
        _conv_body,
        grid=grid,
        in_specs=[
            pl.BlockSpec((BLK, D), lambda i: (i, 0)),
            pl.BlockSpec((BLK, D), lambda i: (i, 0)),
            pl.BlockSpec((BLK, D), lambda i: (i, 0)),
            pl.BlockSpec((D, D), lambda i: (0, 0)),
            pl.BlockSpec((BLK, D), lambda i: (i, 0)),
            pl.BlockSpec(memory_space=pltpu.SMEM),
        ],
        out_specs=[
            pl.BlockSpec((BLK, 2 * D), lambda i: (i, 0)),
            pl.BlockSpec((8, 2 * D), lambda i: (0, 0)),
        ],
        out_shape=[
            jax.ShapeDtypeStruct((N, 2 * D), jnp.float32),
            jax.ShapeDtypeStruct((8, 2 * D), jnp.float32),
        ],
    )(q0, q1, hn, w, nrm, params)


def _radius_body(hc_ref, cs_ref, rr_ref, out_ref):
    o = cs_ref[0:1, :] * (1.0 / N)
    d = hc_ref[...] - o + 1e-6
    lane = lax.broadcasted_iota(jnp.int32, d.shape, 1)
    m = ((lane % D) < (D - 2)).astype(jnp.float32)
    d = d * m
    r = jnp.sqrt(jnp.sum(d * d, axis=1, keepdims=True))
    out_ref[...] = jnp.clip(r - rr_ref[0], 0.0001, 1.0 - 0.0001)


def _tc_radius(hc, cs, ref_radius):
    grid = (N // BLK,)
    return pl.pallas_call(
        _radius_body,
        grid=grid,
        in_specs=[
            pl.BlockSpec((BLK, 2 * D), lambda i: (i, 0)),
            pl.BlockSpec((8, 2 * D), lambda i: (0, 0)),
            pl.BlockSpec(memory_space=pltpu.SMEM),
        ],
        out_specs=pl.BlockSpec((BLK, 1), lambda i: (i, 0)),
        out_shape=jax.ShapeDtypeStruct((N, 1), jnp.float32),
    )(hc, cs, ref_radius)


def kernel(feat, edge_index, W1, W2, conv_w, conv_b, ref_radius):
    src = edge_index[0].reshape(NW, SUPER, IB, CHUNK)
    dst = edge_index[1].reshape(NW, SUPER, IB, CHUNK)
    ei = jnp.stack([src, dst], axis=2).reshape(NW, SUPER, 2 * IB, CHUNK)
    ones128 = jnp.ones((CHUNK, D), jnp.float32)
    zeros128 = jnp.zeros((CHUNK, D), jnp.float32)
    params = jnp.concatenate([conv_w.reshape(6), conv_b]).astype(jnp.float32)

    f1 = _tc_matmul(feat.astype(jnp.float32), W1)
    degp = _sc_degree(dst, ones128, zeros128)
    f1n, nrm = _tc_prep(degp[0], degp[1], f1)

    p = _sc_aggregate(f1n, ei, zeros128)
    hn2 = _tc_layer(p[0], p[1], f1n, nrm)

    q = _sc_aggregate(hn2, ei, zeros128)
    hc, cs = _tc_conv(q[0], q[1], hn2, W2, nrm, params)

    dis = _tc_radius(hc, cs, ref_radius)
    return (dis.reshape(N), ref_radius)
